# baseline (speedup 1.0000x reference)
import functools

import jax
import jax.numpy as jnp
from jax import lax
from jax.experimental import pallas as pl
from jax.experimental.pallas import tpu as pltpu
from jax.experimental.pallas import tpu_sc as plsc

VOCAB = 1000000
N_EMBD = 1024
R = 128

NUM_CORES = 2
NUM_SUBCORES = 16
NW = NUM_CORES * NUM_SUBCORES

def _make_gather(B):
  assert B % (8 * NW) == 0
  b_per_w = B // NW
  CHUNK = max(c for c in range(8, 129, 8) if b_per_w % c == 0)
  assert b_per_w % CHUNK == 0
  n_chunks = b_per_w // CHUNK
  mesh = plsc.VectorSubcoreMesh(core_axis_name="c", subcore_axis_name="s")

  @functools.partial(
      pl.kernel,
      out_type=jax.ShapeDtypeStruct((B, R), jnp.float32),
      mesh=mesh,
      compiler_params=pltpu.CompilerParams(use_tc_tiling_on_sc=True),
      scratch_types=[
          pltpu.VMEM((b_per_w,), jnp.int32),
          pltpu.VMEM((CHUNK, R), jnp.float32),
          pltpu.SemaphoreType.DMA,
      ],
  )
  def gather(table_hbm, idx_hbm, out_hbm, idx_v, rows_v, gsem):
    wid = lax.axis_index("s") * NUM_CORES + lax.axis_index("c")
    base = wid * b_per_w
    pltpu.sync_copy(idx_hbm.at[pl.ds(base, b_per_w)], idx_v)
    for c in range(n_chunks):
      pltpu.async_copy(table_hbm.at[idx_v.at[pl.ds(c * CHUNK, CHUNK)]],
                       rows_v, gsem).wait()
      pltpu.sync_copy(rows_v, out_hbm.at[pl.ds(base + c * CHUNK, CHUNK)])

  return gather


def _proj_body(L, Lp, G, h_ref, w_ref, o_ref):
  y2 = lax.dot_general(
      h_ref[...], w_ref[...],
      dimension_numbers=(((1,), (1,)), ((), ())),
      preferred_element_type=jnp.float32)
  o_ref[...] = y2.reshape(G, Lp, N_EMBD)[:, :L, :]


def _proj_body_alias(L, Lp, G, y_any, h_ref, w_ref, o_ref):
  del y_any
  _proj_body(L, Lp, G, h_ref, w_ref, o_ref)


def _project_phase(y_prev, h, w, Bo, L, Lp, seq0, nseq, G):
  assert nseq % G == 0 and seq0 % G == 0
  out_shape = jax.ShapeDtypeStruct((Bo, L, N_EMBD), jnp.float32)
  p0 = seq0 // G
  common = dict(
      grid=(nseq // G,),
      out_specs=pl.BlockSpec((G, L, N_EMBD), lambda i: (i + p0, 0, 0)),
      out_shape=out_shape,
      compiler_params=pltpu.CompilerParams(
          dimension_semantics=("parallel",)),
  )
  h_spec = pl.BlockSpec((G * Lp, R), lambda i: (i, 0))
  w_spec = pl.BlockSpec((N_EMBD, R), lambda i: (0, 0))
  if y_prev is None:
    return pl.pallas_call(
        functools.partial(_proj_body, L, Lp, G),
        in_specs=[h_spec, w_spec],
        **common,
    )(h, w)
  return pl.pallas_call(
      functools.partial(_proj_body_alias, L, Lp, G),
      in_specs=[pl.BlockSpec(memory_space=pl.ANY), h_spec, w_spec],
      input_output_aliases={0: 0},
      **common,
  )(y_prev, h, w)


N_PHASES = 4


def kernel(x, embed_in_weight, embed_out_weight):
  Bo, L = x.shape
  Lp = -(-L // 8) * 8
  xi = x.astype(jnp.int32)
  xp = jnp.pad(xi, ((0, 0), (0, Lp - L)))
  nseq = Bo // N_PHASES
  gather = _make_gather(nseq * Lp)
  hs = [gather(embed_in_weight, xp[p * nseq:(p + 1) * nseq].reshape(-1))
        for p in range(N_PHASES)]
  y = None
  for p in range(N_PHASES):
    y = _project_phase(y, hs[p], embed_out_weight, Bo, L, Lp,
                       seq0=p * nseq, nseq=nseq, G=64)
  return y


# speedup vs baseline: 1.8986x; 1.8986x over previous
import functools

import jax
import jax.numpy as jnp
from jax import lax
from jax.experimental import pallas as pl
from jax.experimental.pallas import tpu as pltpu
from jax.experimental.pallas import tpu_sc as plsc

VOCAB = 1000000
N_EMBD = 1024
R = 128

NUM_CORES = 2
NUM_SUBCORES = 16
NW = NUM_CORES * NUM_SUBCORES

def _make_gather(B):
  assert B % (8 * NW) == 0
  b_per_w = B // NW
  CHUNK = max(c for c in range(8, 81, 8) if b_per_w % c == 0)
  assert b_per_w % CHUNK == 0
  n_chunks = b_per_w // CHUNK
  mesh = plsc.VectorSubcoreMesh(core_axis_name="c", subcore_axis_name="s")

  @functools.partial(
      pl.kernel,
      out_type=jax.ShapeDtypeStruct((B, R), jnp.float32),
      mesh=mesh,
      compiler_params=pltpu.CompilerParams(use_tc_tiling_on_sc=True),
      scratch_types=[
          pltpu.VMEM((b_per_w,), jnp.int32),
          pltpu.VMEM((CHUNK, R), jnp.float32),
          pltpu.SemaphoreType.DMA,
      ],
  )
  def gather(table_hbm, idx_hbm, out_hbm, idx_v, rows_v, gsem):
    wid = lax.axis_index("s") * NUM_CORES + lax.axis_index("c")
    base = wid * b_per_w
    pltpu.sync_copy(idx_hbm.at[pl.ds(base, b_per_w)], idx_v)
    for c in range(n_chunks):
      pltpu.async_copy(table_hbm.at[idx_v.at[pl.ds(c * CHUNK, CHUNK)]],
                       rows_v, gsem).wait()
      pltpu.sync_copy(rows_v, out_hbm.at[pl.ds(base + c * CHUNK, CHUNK)])

  return gather


def _proj_body(L, Lp, G, h_ref, w_ref, o_ref):
  y2 = lax.dot_general(
      h_ref[...], w_ref[...],
      dimension_numbers=(((1,), (1,)), ((), ())),
      preferred_element_type=jnp.float32)
  o_ref[...] = y2.reshape(G, Lp, N_EMBD)[:, :L, :]


def _proj_body_alias(L, Lp, G, y_any, h_ref, w_ref, o_ref):
  del y_any
  _proj_body(L, Lp, G, h_ref, w_ref, o_ref)


def _project_phase(y_prev, h, w, Bo, L, Lp, seq0, nseq, G):
  assert nseq % G == 0 and seq0 % G == 0
  out_shape = jax.ShapeDtypeStruct((Bo, L, N_EMBD), jnp.float32)
  p0 = seq0 // G
  common = dict(
      grid=(nseq // G,),
      out_specs=pl.BlockSpec((G, L, N_EMBD), lambda i: (i + p0, 0, 0)),
      out_shape=out_shape,
      compiler_params=pltpu.CompilerParams(
          dimension_semantics=("parallel",)),
  )
  h_spec = pl.BlockSpec((G * Lp, R), lambda i: (i, 0))
  w_spec = pl.BlockSpec((N_EMBD, R), lambda i: (0, 0))
  if y_prev is None:
    return pl.pallas_call(
        functools.partial(_proj_body, L, Lp, G),
        in_specs=[h_spec, w_spec],
        **common,
    )(h, w)
  return pl.pallas_call(
      functools.partial(_proj_body_alias, L, Lp, G),
      in_specs=[pl.BlockSpec(memory_space=pl.ANY), h_spec, w_spec],
      input_output_aliases={0: 0},
      **common,
  )(y_prev, h, w)


N_PHASES = 4


def kernel(x, embed_in_weight, embed_out_weight):
  Bo, L = x.shape
  Lp = -(-L // 8) * 8
  xi = x.astype(jnp.int32)
  pad_idx = (jnp.arange(Bo * (Lp - L), dtype=jnp.int32) % VOCAB).reshape(
      Bo, Lp - L)
  xp = jnp.concatenate([xi, pad_idx], axis=1)
  nseq = Bo // N_PHASES
  gather = _make_gather(nseq * Lp)
  hs = [gather(embed_in_weight, xp[p * nseq:(p + 1) * nseq].reshape(-1))
        for p in range(N_PHASES)]
  y = None
  for p in range(N_PHASES):
    y = _project_phase(y, hs[p], embed_out_weight, Bo, L, Lp,
                       seq0=p * nseq, nseq=nseq, G=64)
  return y
